# baseline (speedup 1.0000x reference)
import functools

import jax
import jax.numpy as jnp
from jax import lax
from jax.experimental import pallas as pl
from jax.experimental.pallas import tpu as pltpu, tpu_sc as plsc

_NC = 2
_NS = 16
_NW = _NC * _NS
_LANES = 16
_CHUNK = 64
_TRIG_ROWS = 64

_SIN_C = (9.99999707e-01, -1.66665772e-01, 8.33255785e-03,
          -1.98125681e-04, 2.70404249e-06, -2.05338748e-08)
_COS_C = (9.99999992e-01, -4.99999918e-01, 4.16665243e-02,
          -1.38879701e-03, 2.47734165e-05, -2.71132935e-07,
          1.73688283e-09)
_INV_2PI = 0.15915494309189535
_PI2_HI = 6.2831855
_PI2_LO = -1.7484555e-07


def _horner(coeffs, q):
    acc = jnp.full((_LANES,), coeffs[-1], jnp.float32)
    for co in coeffs[-2::-1]:
        acc = acc * q + jnp.float32(co)
    return acc


def _pack_trig(theta):
    half = jnp.where(theta >= 0, jnp.float32(0.5), jnp.float32(-0.5))
    k = (theta * jnp.float32(_INV_2PI) + half).astype(jnp.int32)
    kf = k.astype(jnp.float32)
    r = (theta - kf * jnp.float32(_PI2_HI)) - kf * jnp.float32(_PI2_LO)
    q = r * r
    s = r * _horner(_SIN_C, q)
    c = _horner(_COS_C, q)
    cb = lax.bitcast_convert_type(c, jnp.int32) + jnp.int32(0x8000)
    sb = lax.bitcast_convert_type(s, jnp.int32) + jnp.int32(0x8000)
    return (cb & jnp.int32(-65536)) | lax.shift_right_logical(sb, 16)


def _sc_score(head_index, rel_type, tail_index, node_emb, node_emb_im,
              rel_emb):
    batch = head_index.shape[0]
    hidden = node_emb.shape[1]
    n_rel = rel_emb.shape[0]
    nslice = hidden // _LANES
    per_w = batch // _NW
    n_chunks = per_w // _CHUNK
    tab_rows = _NS * _TRIG_ROWS
    last_start = n_rel - _TRIG_ROWS
    mesh = plsc.VectorSubcoreMesh(
        core_axis_name="c", subcore_axis_name="s",
        num_cores=_NC, num_subcores=_NS,
    )

    @functools.partial(
        pl.kernel,
        out_type=(jax.ShapeDtypeStruct((batch,), jnp.float32),
                  jax.ShapeDtypeStruct((n_rel, hidden), jnp.int32)),
        mesh=mesh,
        scratch_types=[
            pltpu.VMEM((2, _CHUNK), jnp.int32),
            pltpu.VMEM((2, _CHUNK), jnp.int32),
            pltpu.VMEM((2, _CHUNK), jnp.int32),
            pltpu.VMEM((2, _CHUNK, hidden), jnp.float32),
            pltpu.VMEM((2, _CHUNK, hidden), jnp.float32),
            pltpu.VMEM((2, _CHUNK, hidden), jnp.float32),
            pltpu.VMEM((2, _CHUNK, hidden), jnp.float32),
            pltpu.VMEM((2, _CHUNK, hidden), jnp.int32),
            pltpu.VMEM((per_w,), jnp.float32),
            pltpu.VMEM((_TRIG_ROWS, hidden), jnp.float32),
            pltpu.VMEM((_TRIG_ROWS, hidden), jnp.int32),
            pltpu.SemaphoreType.DMA,
            pltpu.SemaphoreType.DMA,
            pltpu.SemaphoreType.DMA,
            pltpu.SemaphoreType.DMA,
        ],
    )
    def score_kernel(head_hbm, rel_hbm, tail_hbm, emb_hbm, embim_hbm,
                     theta_hbm, out_hbm, cs_tab_hbm, idx_h, idx_r, idx_t,
                     hre, him, tre, tim, cs, score, theta_v, packed_v,
                     sem_i0, sem_i1, sem_g0, sem_g1):
        sem_i = (sem_i0, sem_i1)
        sem_g = (sem_g0, sem_g1)
        sid = lax.axis_index("s")
        wid = sid * _NC + lax.axis_index("c")
        base = wid * per_w
        lane_iota = lax.iota(jnp.int32, _LANES)
        perm_idx = [lax.iota(jnp.int32, _LANES) ^ jnp.int32(d)
                    for d in (1, 2, 4, 8)]
        gdims = lax.GatherDimensionNumbers(
            offset_dims=(), collapsed_slice_dims=(0,), start_index_map=(0,))

        def _lperm(x, pidx):
            return lax.gather(x, pidx[:, None], gdims, (1,),
                              mode=lax.GatherScatterMode.PROMISE_IN_BOUNDS)

        def fire_idx(c):
            p = c & 1
            cbase = base + c * _CHUNK
            return (
                pltpu.async_copy(head_hbm.at[pl.ds(cbase, _CHUNK)],
                                 idx_h.at[p], sem_i[p]),
                pltpu.async_copy(rel_hbm.at[pl.ds(cbase, _CHUNK)],
                                 idx_r.at[p], sem_i[p]),
                pltpu.async_copy(tail_hbm.at[pl.ds(cbase, _CHUNK)],
                                 idx_t.at[p], sem_i[p]),
            )

        def fire_nodes(c):
            p = c & 1
            return (
                pltpu.async_copy(emb_hbm.at[idx_h.at[p]], hre.at[p], sem_g[p]),
                pltpu.async_copy(embim_hbm.at[idx_h.at[p]], him.at[p], sem_g[p]),
                pltpu.async_copy(emb_hbm.at[idx_t.at[p]], tre.at[p], sem_g[p]),
                pltpu.async_copy(embim_hbm.at[idx_t.at[p]], tim.at[p], sem_g[p]),
            )

        def fire_cs(c):
            p = c & 1
            return (
                pltpu.async_copy(cs_tab_hbm.at[idx_r.at[p]], cs.at[p],
                                 sem_g[p]),
            )

        def drain(cps):
            for cp in cps:
                cp.wait()

        idx_cps = {0: fire_idx(0)}
        drain(idx_cps[0])
        gat_cps = {0: fire_nodes(0)}
        idx_cps[1] = fire_idx(1)

        tstart = jnp.minimum(sid * _TRIG_ROWS, jnp.int32(last_start))
        pltpu.sync_copy(theta_hbm.at[pl.ds(tstart, _TRIG_ROWS)], theta_v)

        def trig_row(t, carry):
            for j in range(nslice):
                sl = pl.ds(j * _LANES, _LANES)
                packed_v[t, sl] = _pack_trig(theta_v[t, sl])
            return carry

        lax.fori_loop(0, _TRIG_ROWS, trig_row, 0)
        pltpu.sync_copy(packed_v, cs_tab_hbm.at[pl.ds(tstart, _TRIG_ROWS)])
        plsc.subcore_barrier()

        gat_cps[0] = gat_cps[0] + fire_cs(0)
        drain(idx_cps[1])
        gat_cps[1] = fire_nodes(1) + fire_cs(1)

        for c in range(n_chunks):
            p = c & 1
            if c + 1 < n_chunks and c + 1 not in gat_cps:
                drain(idx_cps[c + 1])
                gat_cps[c + 1] = fire_nodes(c + 1) + fire_cs(c + 1)
            drain(gat_cps[c])
            if c + 2 < n_chunks:
                idx_cps[c + 2] = fire_idx(c + 2)

            def gbody(g, carry):
                def tbody(t, res):
                    row = g * _LANES + t
                    acc = jnp.zeros((_LANES,), jnp.float32)
                    for j in range(nslice):
                        sl = pl.ds(j * _LANES, _LANES)
                        w = cs[p, row, sl]
                        cv = lax.bitcast_convert_type(
                            w & jnp.int32(-65536), jnp.float32)
                        sv = lax.bitcast_convert_type(
                            lax.shift_left(w, 16), jnp.float32)
                        a = hre[p, row, sl]
                        b = him[p, row, sl]
                        u = tre[p, row, sl]
                        v = tim[p, row, sl]
                        re = cv * a - sv * b - u
                        im = cv * b + sv * a - v
                        acc = acc + (re * re + im * im)
                    for pidx in perm_idx:
                        acc = acc + _lperm(acc, pidx)
                    return jnp.where(lane_iota == t, acc, res)

                s2 = lax.fori_loop(0, _LANES, tbody,
                                   jnp.zeros((_LANES,), jnp.float32))
                x = jnp.maximum(s2, jnp.float32(1e-12))
                bits = lax.bitcast_convert_type(x, jnp.int32)
                bits = jnp.int32(0x5F3759DF) - lax.shift_right_logical(bits, 1)
                y = lax.bitcast_convert_type(bits, jnp.float32)
                for _ in range(3):
                    y = y * (jnp.float32(1.5) - jnp.float32(0.5) * x * y * y)
                score[pl.ds(c * _CHUNK + g * _LANES, _LANES)] = -(x * y)
                return carry

            lax.fori_loop(0, _CHUNK // _LANES, gbody, 0)

        pltpu.sync_copy(score, out_hbm.at[pl.ds(base, per_w)])

    score, _ = score_kernel(head_index, rel_type, tail_index,
                            node_emb, node_emb_im, rel_emb)
    return score


def kernel(head_index, rel_type, tail_index, node_emb, node_emb_im, rel_emb):
    return _sc_score(head_index, rel_type, tail_index,
                     node_emb, node_emb_im, rel_emb)


# speedup vs baseline: 1.0188x; 1.0188x over previous
import functools

import jax
import jax.numpy as jnp
from jax import lax
from jax.experimental import pallas as pl
from jax.experimental.pallas import tpu as pltpu, tpu_sc as plsc

_NC = 2
_NS = 16
_NW = _NC * _NS
_LANES = 16
_CHUNK = 64
_TRIG_ROWS = 64

_SIN_C = (9.99999707e-01, -1.66665772e-01, 8.33255785e-03,
          -1.98125681e-04, 2.70404249e-06, -2.05338748e-08)
_COS_C = (9.99999992e-01, -4.99999918e-01, 4.16665243e-02,
          -1.38879701e-03, 2.47734165e-05, -2.71132935e-07,
          1.73688283e-09)
_INV_2PI = 0.15915494309189535
_PI2_HI = 6.2831855
_PI2_LO = -1.7484555e-07


def _horner(coeffs, q):
    acc = jnp.full((_LANES,), coeffs[-1], jnp.float32)
    for co in coeffs[-2::-1]:
        acc = acc * q + jnp.float32(co)
    return acc


def _pack_trig(theta):
    half = jnp.where(theta >= 0, jnp.float32(0.5), jnp.float32(-0.5))
    k = (theta * jnp.float32(_INV_2PI) + half).astype(jnp.int32)
    kf = k.astype(jnp.float32)
    r = (theta - kf * jnp.float32(_PI2_HI)) - kf * jnp.float32(_PI2_LO)
    q = r * r
    s = r * _horner(_SIN_C, q)
    c = _horner(_COS_C, q)
    cb = lax.bitcast_convert_type(c, jnp.int32) + jnp.int32(0x8000)
    sb = lax.bitcast_convert_type(s, jnp.int32) + jnp.int32(0x8000)
    return (cb & jnp.int32(-65536)) | lax.shift_right_logical(sb, 16)


def _sc_score(head_index, rel_type, tail_index, node_emb, node_emb_im,
              rel_emb):
    batch = head_index.shape[0]
    hidden = node_emb.shape[1]
    n_rel = rel_emb.shape[0]
    nslice = hidden // _LANES
    per_w = batch // _NW
    n_chunks = per_w // _CHUNK
    tab_rows = _NS * _TRIG_ROWS
    last_start = n_rel - _TRIG_ROWS
    mesh = plsc.VectorSubcoreMesh(
        core_axis_name="c", subcore_axis_name="s",
        num_cores=_NC, num_subcores=_NS,
    )

    @functools.partial(
        pl.kernel,
        out_type=(jax.ShapeDtypeStruct((batch,), jnp.float32),
                  jax.ShapeDtypeStruct((n_rel, hidden), jnp.int32)),
        mesh=mesh,
        scratch_types=[
            pltpu.VMEM((2, _CHUNK), jnp.int32),
            pltpu.VMEM((2, _CHUNK), jnp.int32),
            pltpu.VMEM((2, _CHUNK), jnp.int32),
            pltpu.VMEM((2, _CHUNK, hidden), jnp.float32),
            pltpu.VMEM((2, _CHUNK, hidden), jnp.float32),
            pltpu.VMEM((2, _CHUNK, hidden), jnp.float32),
            pltpu.VMEM((2, _CHUNK, hidden), jnp.float32),
            pltpu.VMEM((2, _CHUNK, hidden), jnp.int32),
            pltpu.VMEM((per_w,), jnp.float32),
            pltpu.VMEM((_TRIG_ROWS, hidden), jnp.float32),
            pltpu.VMEM((_TRIG_ROWS, hidden), jnp.int32),
            pltpu.SemaphoreType.DMA,
            pltpu.SemaphoreType.DMA,
            pltpu.SemaphoreType.DMA,
            pltpu.SemaphoreType.DMA,
            pltpu.SemaphoreType.DMA,
        ],
    )
    def score_kernel(head_hbm, rel_hbm, tail_hbm, emb_hbm, embim_hbm,
                     theta_hbm, out_hbm, cs_tab_hbm, idx_h, idx_r, idx_t,
                     hre, him, tre, tim, cs, score, theta_v, packed_v,
                     sem_i0, sem_i1, sem_g0, sem_g1, sem_t):
        sem_i = (sem_i0, sem_i1)
        sem_g = (sem_g0, sem_g1)
        sid = lax.axis_index("s")
        wid = sid * _NC + lax.axis_index("c")
        base = wid * per_w
        lane_iota = lax.iota(jnp.int32, _LANES)
        perm_idx = [lax.iota(jnp.int32, _LANES) ^ jnp.int32(d)
                    for d in (1, 2, 4, 8)]
        gdims = lax.GatherDimensionNumbers(
            offset_dims=(), collapsed_slice_dims=(0,), start_index_map=(0,))

        def _lperm(x, pidx):
            return lax.gather(x, pidx[:, None], gdims, (1,),
                              mode=lax.GatherScatterMode.PROMISE_IN_BOUNDS)

        def fire_idx(c):
            p = c & 1
            cbase = base + c * _CHUNK
            return (
                pltpu.async_copy(head_hbm.at[pl.ds(cbase, _CHUNK)],
                                 idx_h.at[p], sem_i[p]),
                pltpu.async_copy(rel_hbm.at[pl.ds(cbase, _CHUNK)],
                                 idx_r.at[p], sem_i[p]),
                pltpu.async_copy(tail_hbm.at[pl.ds(cbase, _CHUNK)],
                                 idx_t.at[p], sem_i[p]),
            )

        def fire_nodes(c):
            p = c & 1
            return (
                pltpu.async_copy(emb_hbm.at[idx_h.at[p]], hre.at[p], sem_g[p]),
                pltpu.async_copy(embim_hbm.at[idx_h.at[p]], him.at[p], sem_g[p]),
                pltpu.async_copy(emb_hbm.at[idx_t.at[p]], tre.at[p], sem_g[p]),
                pltpu.async_copy(embim_hbm.at[idx_t.at[p]], tim.at[p], sem_g[p]),
            )

        def fire_cs(c):
            p = c & 1
            return (
                pltpu.async_copy(cs_tab_hbm.at[idx_r.at[p]], cs.at[p],
                                 sem_g[p]),
            )

        def drain(cps):
            for cp in cps:
                cp.wait()

        tstart = jnp.minimum(sid * _TRIG_ROWS, jnp.int32(last_start))
        theta_cp = pltpu.async_copy(
            theta_hbm.at[pl.ds(tstart, _TRIG_ROWS)], theta_v, sem_t)
        idx_cps = {0: fire_idx(0)}
        drain(idx_cps[0])
        gat_cps = {0: fire_nodes(0)}
        idx_cps[1] = fire_idx(1)
        drain(idx_cps[1])
        gat_cps[1] = fire_nodes(1)
        theta_cp.wait()

        def trig_row(t, carry):
            for j in range(nslice):
                sl = pl.ds(j * _LANES, _LANES)
                packed_v[t, sl] = _pack_trig(theta_v[t, sl])
            return carry

        lax.fori_loop(0, _TRIG_ROWS, trig_row, 0)
        pltpu.sync_copy(packed_v, cs_tab_hbm.at[pl.ds(tstart, _TRIG_ROWS)])
        plsc.subcore_barrier()

        gat_cps[0] = gat_cps[0] + fire_cs(0)
        gat_cps[1] = gat_cps[1] + fire_cs(1)

        for c in range(n_chunks):
            p = c & 1
            if c + 1 < n_chunks and c + 1 not in gat_cps:
                drain(idx_cps[c + 1])
                gat_cps[c + 1] = fire_nodes(c + 1) + fire_cs(c + 1)
            drain(gat_cps[c])
            if c + 2 < n_chunks:
                idx_cps[c + 2] = fire_idx(c + 2)

            def gbody(g, carry):
                def tbody(t, res):
                    row = g * _LANES + t
                    acc = jnp.zeros((_LANES,), jnp.float32)
                    for j in range(nslice):
                        sl = pl.ds(j * _LANES, _LANES)
                        w = cs[p, row, sl]
                        cv = lax.bitcast_convert_type(
                            w & jnp.int32(-65536), jnp.float32)
                        sv = lax.bitcast_convert_type(
                            lax.shift_left(w, 16), jnp.float32)
                        a = hre[p, row, sl]
                        b = him[p, row, sl]
                        u = tre[p, row, sl]
                        v = tim[p, row, sl]
                        re = cv * a - sv * b - u
                        im = cv * b + sv * a - v
                        acc = acc + (re * re + im * im)
                    for pidx in perm_idx:
                        acc = acc + _lperm(acc, pidx)
                    return jnp.where(lane_iota == t, acc, res)

                s2 = lax.fori_loop(0, _LANES, tbody,
                                   jnp.zeros((_LANES,), jnp.float32))
                x = jnp.maximum(s2, jnp.float32(1e-12))
                bits = lax.bitcast_convert_type(x, jnp.int32)
                bits = jnp.int32(0x5F3759DF) - lax.shift_right_logical(bits, 1)
                y = lax.bitcast_convert_type(bits, jnp.float32)
                for _ in range(3):
                    y = y * (jnp.float32(1.5) - jnp.float32(0.5) * x * y * y)
                score[pl.ds(c * _CHUNK + g * _LANES, _LANES)] = -(x * y)
                return carry

            lax.fori_loop(0, _CHUNK // _LANES, gbody, 0)

        pltpu.sync_copy(score, out_hbm.at[pl.ds(base, per_w)])

    score, _ = score_kernel(head_index, rel_type, tail_index,
                            node_emb, node_emb_im, rel_emb)
    return score


def kernel(head_index, rel_type, tail_index, node_emb, node_emb_im, rel_emb):
    return _sc_score(head_index, rel_type, tail_index,
                     node_emb, node_emb_im, rel_emb)


# speedup vs baseline: 1.1021x; 1.0817x over previous
import functools

import jax
import jax.numpy as jnp
from jax import lax
from jax.experimental import pallas as pl
from jax.experimental.pallas import tpu as pltpu, tpu_sc as plsc

_NC = 2
_NS = 16
_NW = _NC * _NS
_LANES = 16
_CHUNK = 64
_TRIG_ROWS = 64

_SIN_C = (9.99999707e-01, -1.66665772e-01, 8.33255785e-03,
          -1.98125681e-04, 2.70404249e-06, -2.05338748e-08)
_COS_C = (9.99999992e-01, -4.99999918e-01, 4.16665243e-02,
          -1.38879701e-03, 2.47734165e-05, -2.71132935e-07,
          1.73688283e-09)
_INV_2PI = 0.15915494309189535
_PI2_HI = 6.2831855
_PI2_LO = -1.7484555e-07


def _horner(coeffs, q):
    acc = jnp.full((_LANES,), coeffs[-1], jnp.float32)
    for co in coeffs[-2::-1]:
        acc = acc * q + jnp.float32(co)
    return acc


def _pack_trig(theta):
    half = jnp.where(theta >= 0, jnp.float32(0.5), jnp.float32(-0.5))
    k = (theta * jnp.float32(_INV_2PI) + half).astype(jnp.int32)
    kf = k.astype(jnp.float32)
    r = (theta - kf * jnp.float32(_PI2_HI)) - kf * jnp.float32(_PI2_LO)
    q = r * r
    s = r * _horner(_SIN_C, q)
    c = _horner(_COS_C, q)
    cb = lax.bitcast_convert_type(c, jnp.int32) + jnp.int32(0x8000)
    sb = lax.bitcast_convert_type(s, jnp.int32) + jnp.int32(0x8000)
    return (cb & jnp.int32(-65536)) | lax.shift_right_logical(sb, 16)


def _sc_score(head_index, rel_type, tail_index, node_emb, node_emb_im,
              rel_emb):
    batch = head_index.shape[0]
    hidden = node_emb.shape[1]
    n_rel = rel_emb.shape[0]
    nslice = hidden // _LANES
    per_w = batch // _NW
    n_chunks = per_w // _CHUNK
    tab_rows = _NS * _TRIG_ROWS
    last_start = n_rel - _TRIG_ROWS
    mesh = plsc.VectorSubcoreMesh(
        core_axis_name="c", subcore_axis_name="s",
        num_cores=_NC, num_subcores=_NS,
    )

    @functools.partial(
        pl.kernel,
        out_type=(jax.ShapeDtypeStruct((batch,), jnp.float32),
                  jax.ShapeDtypeStruct((n_rel, hidden), jnp.int32)),
        mesh=mesh,
        scratch_types=[
            pltpu.VMEM((2, _CHUNK), jnp.int32),
            pltpu.VMEM((2, _CHUNK), jnp.int32),
            pltpu.VMEM((2, _CHUNK), jnp.int32),
            pltpu.VMEM((2, _CHUNK, hidden), jnp.float32),
            pltpu.VMEM((2, _CHUNK, hidden), jnp.float32),
            pltpu.VMEM((2, _CHUNK, hidden), jnp.float32),
            pltpu.VMEM((2, _CHUNK, hidden), jnp.float32),
            pltpu.VMEM((2, _CHUNK, hidden), jnp.int32),
            pltpu.VMEM((per_w,), jnp.float32),
            pltpu.VMEM((_TRIG_ROWS, hidden), jnp.float32),
            pltpu.VMEM((_TRIG_ROWS, hidden), jnp.int32),
            pltpu.SemaphoreType.DMA,
            pltpu.SemaphoreType.DMA,
            pltpu.SemaphoreType.DMA,
            pltpu.SemaphoreType.DMA,
            pltpu.SemaphoreType.DMA,
        ],
    )
    def score_kernel(head_hbm, rel_hbm, tail_hbm, emb_hbm, embim_hbm,
                     theta_hbm, out_hbm, cs_tab_hbm, idx_h, idx_r, idx_t,
                     hre, him, tre, tim, cs, score, theta_v, packed_v,
                     sem_i0, sem_i1, sem_g0, sem_g1, sem_t):
        sem_i = (sem_i0, sem_i1)
        sem_g = (sem_g0, sem_g1)
        sid = lax.axis_index("s")
        wid = sid * _NC + lax.axis_index("c")
        base = wid * per_w
        lane_iota = lax.iota(jnp.int32, _LANES)
        perm_idx = [lax.iota(jnp.int32, _LANES) ^ jnp.int32(d)
                    for d in (1, 2, 4, 8)]
        gdims = lax.GatherDimensionNumbers(
            offset_dims=(), collapsed_slice_dims=(0,), start_index_map=(0,))

        def _lperm(x, pidx):
            return lax.gather(x, pidx[:, None], gdims, (1,),
                              mode=lax.GatherScatterMode.PROMISE_IN_BOUNDS)

        def fire_idx(c, p):
            cbase = base + c * _CHUNK
            return (
                pltpu.async_copy(head_hbm.at[pl.ds(cbase, _CHUNK)],
                                 idx_h.at[p], sem_i[p]),
                pltpu.async_copy(rel_hbm.at[pl.ds(cbase, _CHUNK)],
                                 idx_r.at[p], sem_i[p]),
                pltpu.async_copy(tail_hbm.at[pl.ds(cbase, _CHUNK)],
                                 idx_t.at[p], sem_i[p]),
            )

        def fire_nodes(c, p):
            return (
                pltpu.async_copy(emb_hbm.at[idx_h.at[p]], hre.at[p], sem_g[p]),
                pltpu.async_copy(embim_hbm.at[idx_h.at[p]], him.at[p], sem_g[p]),
                pltpu.async_copy(emb_hbm.at[idx_t.at[p]], tre.at[p], sem_g[p]),
                pltpu.async_copy(embim_hbm.at[idx_t.at[p]], tim.at[p], sem_g[p]),
            )

        def fire_cs(c, p):
            return (
                pltpu.async_copy(cs_tab_hbm.at[idx_r.at[p]], cs.at[p],
                                 sem_g[p]),
            )

        def drain(cps):
            for cp in cps:
                cp.wait()

        def drain_idx(c, p):
            cbase = base + c * _CHUNK
            pltpu.make_async_copy(head_hbm.at[pl.ds(cbase, _CHUNK)],
                                  idx_h.at[p], sem_i[p]).wait()
            pltpu.make_async_copy(rel_hbm.at[pl.ds(cbase, _CHUNK)],
                                  idx_r.at[p], sem_i[p]).wait()
            pltpu.make_async_copy(tail_hbm.at[pl.ds(cbase, _CHUNK)],
                                  idx_t.at[p], sem_i[p]).wait()

        def drain_gat(c, p):
            pltpu.make_async_copy(emb_hbm.at[idx_h.at[p]], hre.at[p],
                                  sem_g[p]).wait()
            pltpu.make_async_copy(embim_hbm.at[idx_h.at[p]], him.at[p],
                                  sem_g[p]).wait()
            pltpu.make_async_copy(emb_hbm.at[idx_t.at[p]], tre.at[p],
                                  sem_g[p]).wait()
            pltpu.make_async_copy(embim_hbm.at[idx_t.at[p]], tim.at[p],
                                  sem_g[p]).wait()
            pltpu.make_async_copy(cs_tab_hbm.at[idx_r.at[p]], cs.at[p],
                                  sem_g[p]).wait()

        def compute(c, s):
            def gbody(g, carry):
                def tbody(t, res):
                    row = g * _LANES + t
                    acc = jnp.zeros((_LANES,), jnp.float32)
                    for j in range(nslice):
                        sl = pl.ds(j * _LANES, _LANES)
                        w = cs[s, row, sl]
                        cv = lax.bitcast_convert_type(
                            w & jnp.int32(-65536), jnp.float32)
                        sv = lax.bitcast_convert_type(
                            lax.shift_left(w, 16), jnp.float32)
                        a = hre[s, row, sl]
                        b = him[s, row, sl]
                        u = tre[s, row, sl]
                        v = tim[s, row, sl]
                        re = cv * a - sv * b - u
                        im = cv * b + sv * a - v
                        acc = acc + (re * re + im * im)
                    for pidx in perm_idx:
                        acc = acc + _lperm(acc, pidx)
                    return jnp.where(lane_iota == t, acc, res)

                s2 = lax.fori_loop(0, _LANES, tbody,
                                   jnp.zeros((_LANES,), jnp.float32))
                x = jnp.maximum(s2, jnp.float32(1e-12))
                bits = lax.bitcast_convert_type(x, jnp.int32)
                bits = jnp.int32(0x5F3759DF) - lax.shift_right_logical(bits, 1)
                y = lax.bitcast_convert_type(bits, jnp.float32)
                for _ in range(3):
                    y = y * (jnp.float32(1.5) - jnp.float32(0.5) * x * y * y)
                score[pl.ds(c * _CHUNK + g * _LANES, _LANES)] = -(x * y)
                return carry

            lax.fori_loop(0, _CHUNK // _LANES, gbody, 0)

        tstart = jnp.minimum(sid * _TRIG_ROWS, jnp.int32(last_start))
        theta_cp = pltpu.async_copy(
            theta_hbm.at[pl.ds(tstart, _TRIG_ROWS)], theta_v, sem_t)
        fire_idx(0, 0)
        drain_idx(0, 0)
        fire_nodes(0, 0)
        fire_idx(1, 1)
        drain_idx(1, 1)
        fire_nodes(1, 1)
        theta_cp.wait()

        def trig_row(t, carry):
            for j in range(nslice):
                sl = pl.ds(j * _LANES, _LANES)
                packed_v[t, sl] = _pack_trig(theta_v[t, sl])
            return carry

        lax.fori_loop(0, _TRIG_ROWS, trig_row, 0)
        pltpu.sync_copy(packed_v, cs_tab_hbm.at[pl.ds(tstart, _TRIG_ROWS)])
        plsc.subcore_barrier()
        fire_cs(0, 0)
        fire_cs(1, 1)

        def pair_body(k, carry):
            a = 2 * k
            b = a + 1
            not_last = k < (n_chunks // 2 - 1)
            drain_gat(a, 0)

            @pl.when(not_last)
            def _():
                fire_idx(a + 2, 0)

            compute(a, 0)

            @pl.when(not_last)
            def _():
                drain_idx(a + 2, 0)
                fire_nodes(a + 2, 0)
                fire_cs(a + 2, 0)

            drain_gat(b, 1)

            @pl.when(not_last)
            def _():
                fire_idx(b + 2, 1)

            compute(b, 1)

            @pl.when(not_last)
            def _():
                drain_idx(b + 2, 1)
                fire_nodes(b + 2, 1)
                fire_cs(b + 2, 1)

            return carry

        lax.fori_loop(0, n_chunks // 2, pair_body, 0)
        pltpu.sync_copy(score, out_hbm.at[pl.ds(base, per_w)])

    score, _ = score_kernel(head_index, rel_type, tail_index,
                            node_emb, node_emb_im, rel_emb)
    return score


def kernel(head_index, rel_type, tail_index, node_emb, node_emb_im, rel_emb):
    return _sc_score(head_index, rel_type, tail_index,
                     node_emb, node_emb_im, rel_emb)


# speedup vs baseline: 1.1214x; 1.0175x over previous
import functools

import jax
import jax.numpy as jnp
from jax import lax
from jax.experimental import pallas as pl
from jax.experimental.pallas import tpu as pltpu, tpu_sc as plsc

_NC = 2
_NS = 16
_NW = _NC * _NS
_LANES = 16
_CHUNK = 64
_TRIG_ROWS = 64

_SIN_C = (9.99449986e-01, -1.65838221e-01, 7.99852030e-03,
          -1.47736456e-04)
_COS_C = (9.99971081e-01, -4.99837540e-01, 4.15222679e-02,
          -1.34409944e-03, 1.90647593e-05)
_INV_2PI = 0.15915494309189535
_PI2_HI = 6.2831855
_PI2_LO = -1.7484555e-07


def _horner(coeffs, q):
    acc = jnp.full((_LANES,), coeffs[-1], jnp.float32)
    for co in coeffs[-2::-1]:
        acc = acc * q + jnp.float32(co)
    return acc


def _pack_trig(theta):
    half = jnp.where(theta >= 0, jnp.float32(0.5), jnp.float32(-0.5))
    k = (theta * jnp.float32(_INV_2PI) + half).astype(jnp.int32)
    kf = k.astype(jnp.float32)
    r = (theta - kf * jnp.float32(_PI2_HI)) - kf * jnp.float32(_PI2_LO)
    q = r * r
    s = r * _horner(_SIN_C, q)
    c = _horner(_COS_C, q)
    cb = lax.bitcast_convert_type(c, jnp.int32) + jnp.int32(0x8000)
    sb = lax.bitcast_convert_type(s, jnp.int32) + jnp.int32(0x8000)
    return (cb & jnp.int32(-65536)) | lax.shift_right_logical(sb, 16)


def _sc_score(head_index, rel_type, tail_index, node_emb, node_emb_im,
              rel_emb):
    batch = head_index.shape[0]
    hidden = node_emb.shape[1]
    n_rel = rel_emb.shape[0]
    nslice = hidden // _LANES
    per_w = batch // _NW
    n_chunks = per_w // _CHUNK
    tab_rows = _NS * _TRIG_ROWS
    last_start = n_rel - _TRIG_ROWS
    mesh = plsc.VectorSubcoreMesh(
        core_axis_name="c", subcore_axis_name="s",
        num_cores=_NC, num_subcores=_NS,
    )

    @functools.partial(
        pl.kernel,
        out_type=(jax.ShapeDtypeStruct((batch,), jnp.float32),
                  jax.ShapeDtypeStruct((n_rel, hidden), jnp.int32)),
        mesh=mesh,
        scratch_types=[
            pltpu.VMEM((2, _CHUNK), jnp.int32),
            pltpu.VMEM((2, _CHUNK), jnp.int32),
            pltpu.VMEM((2, _CHUNK), jnp.int32),
            pltpu.VMEM((2, _CHUNK, hidden), jnp.float32),
            pltpu.VMEM((2, _CHUNK, hidden), jnp.float32),
            pltpu.VMEM((2, _CHUNK, hidden), jnp.float32),
            pltpu.VMEM((2, _CHUNK, hidden), jnp.float32),
            pltpu.VMEM((2, _CHUNK, hidden), jnp.int32),
            pltpu.VMEM((per_w,), jnp.float32),
            pltpu.VMEM((_TRIG_ROWS, hidden), jnp.float32),
            pltpu.VMEM((_TRIG_ROWS, hidden), jnp.int32),
            pltpu.SemaphoreType.DMA,
            pltpu.SemaphoreType.DMA,
            pltpu.SemaphoreType.DMA,
            pltpu.SemaphoreType.DMA,
            pltpu.SemaphoreType.DMA,
        ],
    )
    def score_kernel(head_hbm, rel_hbm, tail_hbm, emb_hbm, embim_hbm,
                     theta_hbm, out_hbm, cs_tab_hbm, idx_h, idx_r, idx_t,
                     hre, him, tre, tim, cs, score, theta_v, packed_v,
                     sem_i0, sem_i1, sem_g0, sem_g1, sem_t):
        sem_i = (sem_i0, sem_i1)
        sem_g = (sem_g0, sem_g1)
        sid = lax.axis_index("s")
        wid = sid * _NC + lax.axis_index("c")
        base = wid * per_w
        lane_iota = lax.iota(jnp.int32, _LANES)
        perm_idx = [lax.iota(jnp.int32, _LANES) ^ jnp.int32(d)
                    for d in (1, 2, 4, 8)]
        gdims = lax.GatherDimensionNumbers(
            offset_dims=(), collapsed_slice_dims=(0,), start_index_map=(0,))

        def _lperm(x, pidx):
            return lax.gather(x, pidx[:, None], gdims, (1,),
                              mode=lax.GatherScatterMode.PROMISE_IN_BOUNDS)

        def fire_idx(c, p):
            cbase = base + c * _CHUNK
            return (
                pltpu.async_copy(head_hbm.at[pl.ds(cbase, _CHUNK)],
                                 idx_h.at[p], sem_i[p]),
                pltpu.async_copy(rel_hbm.at[pl.ds(cbase, _CHUNK)],
                                 idx_r.at[p], sem_i[p]),
                pltpu.async_copy(tail_hbm.at[pl.ds(cbase, _CHUNK)],
                                 idx_t.at[p], sem_i[p]),
            )

        def fire_nodes(c, p):
            return (
                pltpu.async_copy(emb_hbm.at[idx_h.at[p]], hre.at[p], sem_g[p]),
                pltpu.async_copy(embim_hbm.at[idx_h.at[p]], him.at[p], sem_g[p]),
                pltpu.async_copy(emb_hbm.at[idx_t.at[p]], tre.at[p], sem_g[p]),
                pltpu.async_copy(embim_hbm.at[idx_t.at[p]], tim.at[p], sem_g[p]),
            )

        def fire_cs(c, p):
            return (
                pltpu.async_copy(cs_tab_hbm.at[idx_r.at[p]], cs.at[p],
                                 sem_g[p]),
            )

        def drain(cps):
            for cp in cps:
                cp.wait()

        def drain_idx(c, p):
            cbase = base + c * _CHUNK
            pltpu.make_async_copy(head_hbm.at[pl.ds(cbase, _CHUNK)],
                                  idx_h.at[p], sem_i[p]).wait()
            pltpu.make_async_copy(rel_hbm.at[pl.ds(cbase, _CHUNK)],
                                  idx_r.at[p], sem_i[p]).wait()
            pltpu.make_async_copy(tail_hbm.at[pl.ds(cbase, _CHUNK)],
                                  idx_t.at[p], sem_i[p]).wait()

        def drain_gat(c, p):
            pltpu.make_async_copy(emb_hbm.at[idx_h.at[p]], hre.at[p],
                                  sem_g[p]).wait()
            pltpu.make_async_copy(embim_hbm.at[idx_h.at[p]], him.at[p],
                                  sem_g[p]).wait()
            pltpu.make_async_copy(emb_hbm.at[idx_t.at[p]], tre.at[p],
                                  sem_g[p]).wait()
            pltpu.make_async_copy(embim_hbm.at[idx_t.at[p]], tim.at[p],
                                  sem_g[p]).wait()
            pltpu.make_async_copy(cs_tab_hbm.at[idx_r.at[p]], cs.at[p],
                                  sem_g[p]).wait()

        def compute(c, s):
            def gbody(g, carry):
                def tbody(t, res):
                    row = g * _LANES + t
                    acc = jnp.zeros((_LANES,), jnp.float32)
                    for j in range(nslice):
                        sl = pl.ds(j * _LANES, _LANES)
                        w = cs[s, row, sl]
                        cv = lax.bitcast_convert_type(
                            w & jnp.int32(-65536), jnp.float32)
                        sv = lax.bitcast_convert_type(
                            lax.shift_left(w, 16), jnp.float32)
                        a = hre[s, row, sl]
                        b = him[s, row, sl]
                        u = tre[s, row, sl]
                        v = tim[s, row, sl]
                        re = cv * a - sv * b - u
                        im = cv * b + sv * a - v
                        acc = acc + (re * re + im * im)
                    for pidx in perm_idx:
                        acc = acc + _lperm(acc, pidx)
                    return jnp.where(lane_iota == t, acc, res)

                s2 = lax.fori_loop(0, _LANES, tbody,
                                   jnp.zeros((_LANES,), jnp.float32))
                x = jnp.maximum(s2, jnp.float32(1e-12))
                bits = lax.bitcast_convert_type(x, jnp.int32)
                bits = jnp.int32(0x5F3759DF) - lax.shift_right_logical(bits, 1)
                y = lax.bitcast_convert_type(bits, jnp.float32)
                for _ in range(3):
                    y = y * (jnp.float32(1.5) - jnp.float32(0.5) * x * y * y)
                score[pl.ds(c * _CHUNK + g * _LANES, _LANES)] = -(x * y)
                return carry

            lax.fori_loop(0, _CHUNK // _LANES, gbody, 0)

        tstart = jnp.minimum(sid * _TRIG_ROWS, jnp.int32(last_start))
        theta_cp = pltpu.async_copy(
            theta_hbm.at[pl.ds(tstart, _TRIG_ROWS)], theta_v, sem_t)
        fire_idx(0, 0)
        drain_idx(0, 0)
        fire_nodes(0, 0)
        fire_idx(1, 1)
        drain_idx(1, 1)
        fire_nodes(1, 1)
        theta_cp.wait()

        def trig_row(t, carry):
            for j in range(nslice):
                sl = pl.ds(j * _LANES, _LANES)
                packed_v[t, sl] = _pack_trig(theta_v[t, sl])
            return carry

        lax.fori_loop(0, _TRIG_ROWS, trig_row, 0)
        pltpu.sync_copy(packed_v, cs_tab_hbm.at[pl.ds(tstart, _TRIG_ROWS)])
        plsc.subcore_barrier()
        fire_cs(0, 0)
        fire_cs(1, 1)

        def pair_body(k, carry):
            a = 2 * k
            b = a + 1
            not_last = k < (n_chunks // 2 - 1)
            drain_gat(a, 0)

            @pl.when(not_last)
            def _():
                fire_idx(a + 2, 0)

            compute(a, 0)

            @pl.when(not_last)
            def _():
                drain_idx(a + 2, 0)
                fire_nodes(a + 2, 0)
                fire_cs(a + 2, 0)

            drain_gat(b, 1)

            @pl.when(not_last)
            def _():
                fire_idx(b + 2, 1)

            compute(b, 1)

            @pl.when(not_last)
            def _():
                drain_idx(b + 2, 1)
                fire_nodes(b + 2, 1)
                fire_cs(b + 2, 1)

            return carry

        lax.fori_loop(0, n_chunks // 2, pair_body, 0)
        pltpu.sync_copy(score, out_hbm.at[pl.ds(base, per_w)])

    score, _ = score_kernel(head_index, rel_type, tail_index,
                            node_emb, node_emb_im, rel_emb)
    return score


def kernel(head_index, rel_type, tail_index, node_emb, node_emb_im, rel_emb):
    return _sc_score(head_index, rel_type, tail_index,
                     node_emb, node_emb_im, rel_emb)


# speedup vs baseline: 1.1254x; 1.0036x over previous
import functools

import jax
import jax.numpy as jnp
from jax import lax
from jax.experimental import pallas as pl
from jax.experimental.pallas import tpu as pltpu, tpu_sc as plsc

_NC = 2
_NS = 16
_NW = _NC * _NS
_LANES = 16
_CHUNK = 64
_TRIG_ROWS = 64

_SIN_C = (9.99449986e-01, -1.65838221e-01, 7.99852030e-03,
          -1.47736456e-04)
_COS_C = (9.99971081e-01, -4.99837540e-01, 4.15222679e-02,
          -1.34409944e-03, 1.90647593e-05)
_INV_2PI = 0.15915494309189535
_PI2_HI = 6.2831855
_PI2_LO = -1.7484555e-07


def _horner(coeffs, q):
    acc = jnp.full((_LANES,), coeffs[-1], jnp.float32)
    for co in coeffs[-2::-1]:
        acc = acc * q + jnp.float32(co)
    return acc


def _pack_trig(theta):
    half = jnp.where(theta >= 0, jnp.float32(0.5), jnp.float32(-0.5))
    k = (theta * jnp.float32(_INV_2PI) + half).astype(jnp.int32)
    kf = k.astype(jnp.float32)
    r = (theta - kf * jnp.float32(_PI2_HI)) - kf * jnp.float32(_PI2_LO)
    q = r * r
    s = r * _horner(_SIN_C, q)
    c = _horner(_COS_C, q)
    cb = lax.bitcast_convert_type(c, jnp.int32) + jnp.int32(0x8000)
    sb = lax.bitcast_convert_type(s, jnp.int32) + jnp.int32(0x8000)
    return (cb & jnp.int32(-65536)) | lax.shift_right_logical(sb, 16)


def _sc_score(head_index, rel_type, tail_index, node_emb, node_emb_im,
              rel_emb):
    batch = head_index.shape[0]
    hidden = node_emb.shape[1]
    n_rel = rel_emb.shape[0]
    nslice = hidden // _LANES
    per_w = batch // _NW
    n_chunks = per_w // _CHUNK
    tab_rows = _NS * _TRIG_ROWS
    last_start = n_rel - _TRIG_ROWS
    mesh = plsc.VectorSubcoreMesh(
        core_axis_name="c", subcore_axis_name="s",
        num_cores=_NC, num_subcores=_NS,
    )

    @functools.partial(
        pl.kernel,
        out_type=(jax.ShapeDtypeStruct((batch,), jnp.float32),
                  jax.ShapeDtypeStruct((n_rel, hidden), jnp.int32)),
        mesh=mesh,
        scratch_types=[
            pltpu.VMEM((n_chunks, 2 * _CHUNK), jnp.int32),
            pltpu.VMEM((n_chunks, _CHUNK), jnp.int32),
            pltpu.VMEM((2, 2 * _CHUNK, hidden), jnp.float32),
            pltpu.VMEM((2, 2 * _CHUNK, hidden), jnp.float32),
            pltpu.VMEM((2, _CHUNK, hidden), jnp.int32),
            pltpu.VMEM((per_w,), jnp.float32),
            pltpu.VMEM((_TRIG_ROWS, hidden), jnp.float32),
            pltpu.VMEM((_TRIG_ROWS, hidden), jnp.int32),
            pltpu.SemaphoreType.DMA,
            pltpu.SemaphoreType.DMA,
            pltpu.SemaphoreType.DMA,
            pltpu.SemaphoreType.DMA,
        ],
    )
    def score_kernel(head_hbm, rel_hbm, tail_hbm, emb_hbm, embim_hbm,
                     theta_hbm, out_hbm, cs_tab_hbm, idx_ht, idx_r2,
                     ht_re, ht_im, cs, score, theta_v, packed_v,
                     sem_i, sem_g0, sem_g1, sem_t):
        sem_g = (sem_g0, sem_g1)
        sid = lax.axis_index("s")
        wid = sid * _NC + lax.axis_index("c")
        base = wid * per_w
        lane_iota = lax.iota(jnp.int32, _LANES)
        perm_idx = [lax.iota(jnp.int32, _LANES) ^ jnp.int32(d)
                    for d in (1, 2, 4, 8)]
        gdims = lax.GatherDimensionNumbers(
            offset_dims=(), collapsed_slice_dims=(0,), start_index_map=(0,))

        def _lperm(x, pidx):
            return lax.gather(x, pidx[:, None], gdims, (1,),
                              mode=lax.GatherScatterMode.PROMISE_IN_BOUNDS)

        def fire_nodes(c, p):
            return (
                pltpu.async_copy(emb_hbm.at[idx_ht.at[c]], ht_re.at[p],
                                 sem_g[p]),
                pltpu.async_copy(embim_hbm.at[idx_ht.at[c]], ht_im.at[p],
                                 sem_g[p]),
            )

        def fire_cs(c, p):
            return (
                pltpu.async_copy(cs_tab_hbm.at[idx_r2.at[c]], cs.at[p],
                                 sem_g[p]),
            )

        def drain_gat(c, p):
            pltpu.make_async_copy(emb_hbm.at[idx_ht.at[c]], ht_re.at[p],
                                  sem_g[p]).wait()
            pltpu.make_async_copy(embim_hbm.at[idx_ht.at[c]], ht_im.at[p],
                                  sem_g[p]).wait()
            pltpu.make_async_copy(cs_tab_hbm.at[idx_r2.at[c]], cs.at[p],
                                  sem_g[p]).wait()

        def compute(c, s):
            def gbody(g, carry):
                def tbody(t, res):
                    row = g * _LANES + t
                    acc = jnp.zeros((_LANES,), jnp.float32)
                    for j in range(nslice):
                        sl = pl.ds(j * _LANES, _LANES)
                        w = cs[s, row, sl]
                        cv = lax.bitcast_convert_type(
                            w & jnp.int32(-65536), jnp.float32)
                        sv = lax.bitcast_convert_type(
                            lax.shift_left(w, 16), jnp.float32)
                        a = ht_re[s, row, sl]
                        b = ht_im[s, row, sl]
                        u = ht_re[s, _CHUNK + row, sl]
                        v = ht_im[s, _CHUNK + row, sl]
                        re = cv * a - sv * b - u
                        im = cv * b + sv * a - v
                        acc = acc + (re * re + im * im)
                    for pidx in perm_idx:
                        acc = acc + _lperm(acc, pidx)
                    return jnp.where(lane_iota == t, acc, res)

                s2 = lax.fori_loop(0, _LANES, tbody,
                                   jnp.zeros((_LANES,), jnp.float32))
                x = jnp.maximum(s2, jnp.float32(1e-12))
                bits = lax.bitcast_convert_type(x, jnp.int32)
                bits = jnp.int32(0x5F3759DF) - lax.shift_right_logical(bits, 1)
                y = lax.bitcast_convert_type(bits, jnp.float32)
                for _ in range(3):
                    y = y * (jnp.float32(1.5) - jnp.float32(0.5) * x * y * y)
                score[pl.ds(c * _CHUNK + g * _LANES, _LANES)] = -(x * y)
                return carry

            lax.fori_loop(0, _CHUNK // _LANES, gbody, 0)

        tstart = jnp.minimum(sid * _TRIG_ROWS, jnp.int32(last_start))
        theta_cp = pltpu.async_copy(
            theta_hbm.at[pl.ds(tstart, _TRIG_ROWS)], theta_v, sem_t)
        idx_cps = []
        for c in range(n_chunks):
            cbase = base + c * _CHUNK
            idx_cps += [
                pltpu.async_copy(head_hbm.at[pl.ds(cbase, _CHUNK)],
                                 idx_ht.at[c, pl.ds(0, _CHUNK)], sem_i),
                pltpu.async_copy(tail_hbm.at[pl.ds(cbase, _CHUNK)],
                                 idx_ht.at[c, pl.ds(_CHUNK, _CHUNK)], sem_i),
                pltpu.async_copy(rel_hbm.at[pl.ds(cbase, _CHUNK)],
                                 idx_r2.at[c], sem_i),
            ]
        for cp in idx_cps:
            cp.wait()
        fire_nodes(0, 0)
        fire_nodes(1, 1)
        theta_cp.wait()

        def trig_row(t, carry):
            for j in range(nslice):
                sl = pl.ds(j * _LANES, _LANES)
                packed_v[t, sl] = _pack_trig(theta_v[t, sl])
            return carry

        lax.fori_loop(0, _TRIG_ROWS, trig_row, 0)
        pltpu.sync_copy(packed_v, cs_tab_hbm.at[pl.ds(tstart, _TRIG_ROWS)])
        plsc.subcore_barrier()
        fire_cs(0, 0)
        fire_cs(1, 1)

        def pair_body(k, carry):
            a = 2 * k
            b = a + 1
            not_last = k < (n_chunks // 2 - 1)
            drain_gat(a, 0)
            compute(a, 0)

            @pl.when(not_last)
            def _():
                fire_nodes(a + 2, 0)
                fire_cs(a + 2, 0)

            drain_gat(b, 1)
            compute(b, 1)

            @pl.when(not_last)
            def _():
                fire_nodes(b + 2, 1)
                fire_cs(b + 2, 1)

            return carry

        lax.fori_loop(0, n_chunks // 2, pair_body, 0)
        pltpu.sync_copy(score, out_hbm.at[pl.ds(base, per_w)])

    score, _ = score_kernel(head_index, rel_type, tail_index,
                            node_emb, node_emb_im, rel_emb)
    return score


def kernel(head_index, rel_type, tail_index, node_emb, node_emb_im, rel_emb):
    return _sc_score(head_index, rel_type, tail_index,
                     node_emb, node_emb_im, rel_emb)


# speedup vs baseline: 1.1476x; 1.0198x over previous
import functools

import jax
import jax.numpy as jnp
from jax import lax
from jax.experimental import pallas as pl
from jax.experimental.pallas import tpu as pltpu, tpu_sc as plsc

_NC = 2
_NS = 16
_NW = _NC * _NS
_LANES = 16
_CHUNK = 64
_TRIG_ROWS = 64

_SIN_C = (9.99449986e-01, -1.65838221e-01, 7.99852030e-03,
          -1.47736456e-04)
_COS_C = (9.99971081e-01, -4.99837540e-01, 4.15222679e-02,
          -1.34409944e-03, 1.90647593e-05)
_INV_2PI = 0.15915494309189535
_PI2_HI = 6.2831855
_PI2_LO = -1.7484555e-07


def _horner(coeffs, q):
    acc = jnp.full((_LANES,), coeffs[-1], jnp.float32)
    for co in coeffs[-2::-1]:
        acc = acc * q + jnp.float32(co)
    return acc


def _pack_trig(theta):
    half = jnp.where(theta >= 0, jnp.float32(0.5), jnp.float32(-0.5))
    k = (theta * jnp.float32(_INV_2PI) + half).astype(jnp.int32)
    kf = k.astype(jnp.float32)
    r = (theta - kf * jnp.float32(_PI2_HI)) - kf * jnp.float32(_PI2_LO)
    q = r * r
    s = r * _horner(_SIN_C, q)
    c = _horner(_COS_C, q)
    cb = lax.bitcast_convert_type(c, jnp.int32) + jnp.int32(0x8000)
    sb = lax.bitcast_convert_type(s, jnp.int32) + jnp.int32(0x8000)
    return (cb & jnp.int32(-65536)) | lax.shift_right_logical(sb, 16)


def _sc_score(head_index, rel_type, tail_index, node_emb, node_emb_im,
              rel_emb):
    batch = head_index.shape[0]
    hidden = node_emb.shape[1]
    n_rel = rel_emb.shape[0]
    nslice = hidden // _LANES
    per_w = batch // _NW
    n_chunks = per_w // _CHUNK
    tab_rows = _NS * _TRIG_ROWS
    last_start = n_rel - _TRIG_ROWS
    mesh = plsc.VectorSubcoreMesh(
        core_axis_name="c", subcore_axis_name="s",
        num_cores=_NC, num_subcores=_NS,
    )

    @functools.partial(
        pl.kernel,
        out_type=(jax.ShapeDtypeStruct((batch,), jnp.float32),
                  jax.ShapeDtypeStruct((n_rel, hidden), jnp.int32)),
        mesh=mesh,
        scratch_types=[
            pltpu.VMEM((n_chunks, 2 * _CHUNK), jnp.int32),
            pltpu.VMEM((n_chunks, _CHUNK), jnp.int32),
            pltpu.VMEM((2, 2 * _CHUNK, hidden), jnp.float32),
            pltpu.VMEM((2, 2 * _CHUNK, hidden), jnp.float32),
            pltpu.VMEM((2, _CHUNK, hidden), jnp.int32),
            pltpu.VMEM((per_w,), jnp.float32),
            pltpu.VMEM((_TRIG_ROWS, hidden), jnp.float32),
            pltpu.VMEM((_TRIG_ROWS, hidden), jnp.int32),
            pltpu.SemaphoreType.DMA,
            pltpu.SemaphoreType.DMA((2,)),
            pltpu.SemaphoreType.DMA,
        ],
    )
    def score_kernel(head_hbm, rel_hbm, tail_hbm, emb_hbm, embim_hbm,
                     theta_hbm, out_hbm, cs_tab_hbm, idx_ht, idx_r2,
                     ht_re, ht_im, cs, score, theta_v, packed_v,
                     sem_i, sem_g, sem_t):
        sid = lax.axis_index("s")
        wid = sid * _NC + lax.axis_index("c")
        base = wid * per_w
        lane_iota = lax.iota(jnp.int32, _LANES)
        perm_idx = [lax.iota(jnp.int32, _LANES) ^ jnp.int32(d)
                    for d in (1, 2, 4, 8)]
        gdims = lax.GatherDimensionNumbers(
            offset_dims=(), collapsed_slice_dims=(0,), start_index_map=(0,))

        def _lperm(x, pidx):
            return lax.gather(x, pidx[:, None], gdims, (1,),
                              mode=lax.GatherScatterMode.PROMISE_IN_BOUNDS)

        def fire_nodes(c, p):
            return (
                pltpu.async_copy(emb_hbm.at[idx_ht.at[c]], ht_re.at[p],
                                 sem_g.at[p]),
                pltpu.async_copy(embim_hbm.at[idx_ht.at[c]], ht_im.at[p],
                                 sem_g.at[p]),
            )

        def fire_cs(c, p):
            return (
                pltpu.async_copy(cs_tab_hbm.at[idx_r2.at[c]], cs.at[p],
                                 sem_g.at[p]),
            )

        def drain_gat(c, p):
            pltpu.make_async_copy(emb_hbm.at[idx_ht.at[c]], ht_re.at[p],
                                  sem_g.at[p]).wait()
            pltpu.make_async_copy(embim_hbm.at[idx_ht.at[c]], ht_im.at[p],
                                  sem_g.at[p]).wait()
            pltpu.make_async_copy(cs_tab_hbm.at[idx_r2.at[c]], cs.at[p],
                                  sem_g.at[p]).wait()

        def compute(c, s):
            def gbody(g, carry):
                def tbody(t, res):
                    row = g * _LANES + t
                    acc = jnp.zeros((_LANES,), jnp.float32)
                    for j in range(nslice):
                        sl = pl.ds(j * _LANES, _LANES)
                        w = cs[s, row, sl]
                        cv = lax.bitcast_convert_type(
                            w & jnp.int32(-65536), jnp.float32)
                        sv = lax.bitcast_convert_type(
                            lax.shift_left(w, 16), jnp.float32)
                        a = ht_re[s, row, sl]
                        b = ht_im[s, row, sl]
                        u = ht_re[s, _CHUNK + row, sl]
                        v = ht_im[s, _CHUNK + row, sl]
                        re = cv * a - sv * b - u
                        im = cv * b + sv * a - v
                        acc = acc + (re * re + im * im)
                    for pidx in perm_idx:
                        acc = acc + _lperm(acc, pidx)
                    return jnp.where(lane_iota == t, acc, res)

                s2 = lax.fori_loop(0, _LANES, tbody,
                                   jnp.zeros((_LANES,), jnp.float32))
                x = jnp.maximum(s2, jnp.float32(1e-12))
                bits = lax.bitcast_convert_type(x, jnp.int32)
                bits = jnp.int32(0x5F3759DF) - lax.shift_right_logical(bits, 1)
                y = lax.bitcast_convert_type(bits, jnp.float32)
                for _ in range(3):
                    y = y * (jnp.float32(1.5) - jnp.float32(0.5) * x * y * y)
                score[pl.ds(c * _CHUNK + g * _LANES, _LANES)] = -(x * y)
                return carry

            lax.fori_loop(0, _CHUNK // _LANES, gbody, 0)

        tstart = jnp.minimum(sid * _TRIG_ROWS, jnp.int32(last_start))
        theta_cp = pltpu.async_copy(
            theta_hbm.at[pl.ds(tstart, _TRIG_ROWS)], theta_v, sem_t)
        idx_cps = []
        for c in range(n_chunks):
            cbase = base + c * _CHUNK
            idx_cps += [
                pltpu.async_copy(head_hbm.at[pl.ds(cbase, _CHUNK)],
                                 idx_ht.at[c, pl.ds(0, _CHUNK)], sem_i),
                pltpu.async_copy(tail_hbm.at[pl.ds(cbase, _CHUNK)],
                                 idx_ht.at[c, pl.ds(_CHUNK, _CHUNK)], sem_i),
                pltpu.async_copy(rel_hbm.at[pl.ds(cbase, _CHUNK)],
                                 idx_r2.at[c], sem_i),
            ]
        for cp in idx_cps:
            cp.wait()
        fire_nodes(0, 0)
        fire_nodes(1, 1)
        theta_cp.wait()

        def trig_row(t, carry):
            for j in range(nslice):
                sl = pl.ds(j * _LANES, _LANES)
                packed_v[t, sl] = _pack_trig(theta_v[t, sl])
            return carry

        lax.fori_loop(0, _TRIG_ROWS, trig_row, 0)
        pltpu.sync_copy(packed_v, cs_tab_hbm.at[pl.ds(tstart, _TRIG_ROWS)])
        plsc.subcore_barrier()
        fire_cs(0, 0)
        fire_cs(1, 1)

        def chunk_body(c, carry):
            p = lax.rem(c, 2)
            drain_gat(c, p)
            compute(c, p)

            @pl.when(c + 2 < n_chunks)
            def _():
                fire_nodes(c + 2, p)
                fire_cs(c + 2, p)

            return carry

        lax.fori_loop(0, n_chunks, chunk_body, 0)
        pltpu.sync_copy(score, out_hbm.at[pl.ds(base, per_w)])

    score, _ = score_kernel(head_index, rel_type, tail_index,
                            node_emb, node_emb_im, rel_emb)
    return score


def kernel(head_index, rel_type, tail_index, node_emb, node_emb_im, rel_emb):
    return _sc_score(head_index, rel_type, tail_index,
                     node_emb, node_emb_im, rel_emb)
